# baseline (speedup 1.0000x reference)
import jax
import jax.numpy as jnp
from jax.experimental import pallas as pl
from jax.experimental.pallas import tpu as pltpu
from jax.experimental.pallas import tpu_sc as plsc

B = 16384
E = 128
FEAT = 64
H1, H2 = 512, 256
OUT = 128

GATHER_WINDOW = 256
BB = 4096


def _make_sc_gather(n):
    mesh = plsc.VectorSubcoreMesh(core_axis_name="c", subcore_axis_name="s")
    out_type = jax.ShapeDtypeStruct((n, E), jnp.float32)

    @pl.kernel(out_type=out_type, mesh=mesh)
    def gather_kernel(t_hbm, i_hbm, o_hbm):
        def body(i_vmem, o_vmem):
            pltpu.sync_copy(t_hbm.at[i_vmem.at[0]], o_vmem)

        pltpu.emit_pipeline(
            body,
            grid=(n // GATHER_WINDOW,),
            in_specs=[pl.BlockSpec((1, GATHER_WINDOW), lambda i: (0, i))],
            out_specs=[pl.BlockSpec((GATHER_WINDOW, E), lambda i: (i, 0))],
            core_axis_name=("c", "s"),
            dimension_semantics=(pltpu.PARALLEL,),
        )(i_hbm, o_hbm)

    return gather_kernel


_GATHER = _make_sc_gather(B)


def _sc_gather(table, idx, n):
    return _GATHER(table, idx.reshape(1, n))


def _tower_body(emb, feat_t, W1, b1, W2, b2, W3, b3, out_ref):
    zero = jnp.bfloat16(0.0)
    xe = emb[...].astype(jnp.bfloat16)
    xft = feat_t[...]
    x = jnp.concatenate([xe, xft.T], axis=-1)
    h = jnp.dot(x, W1[...], preferred_element_type=jnp.float32)
    h = jnp.maximum(h.astype(jnp.bfloat16) + b1[...], zero)
    h = jnp.dot(h, W2[...], preferred_element_type=jnp.float32)
    h = jnp.maximum(h.astype(jnp.bfloat16) + b2[...], zero)
    o = jnp.dot(h, W3[...], preferred_element_type=jnp.float32) + b3[...]
    ss = jnp.sum(o * o, axis=-1, keepdims=True)
    out_ref[...] = o * jax.lax.rsqrt(jnp.maximum(ss, 1e-24))


def _tc_tower(emb, feats_t, weights, n):
    nb = n // BB
    bspec = lambda shape: pl.BlockSpec(shape, lambda i: (i, 0))
    tspec = lambda shape: pl.BlockSpec(shape, lambda i: (0, i))
    wspec = lambda shape: pl.BlockSpec(shape, lambda i: (0, 0))
    in_specs = [
        bspec((BB, E)), tspec((FEAT, BB)),
        wspec((E + FEAT, H1)), wspec((1, H1)),
        wspec((H1, H2)), wspec((1, H2)),
        wspec((H2, OUT)), wspec((1, OUT)),
    ]
    return pl.pallas_call(
        _tower_body,
        grid=(nb,),
        in_specs=in_specs,
        out_specs=bspec((BB, OUT)),
        out_shape=jax.ShapeDtypeStruct((n, OUT), jnp.float32),
    )(emb, feats_t, *weights)


def kernel(user_idx, user_feats, item_idx, item_feats, user_table, item_table,
           uW1, ub1, uW2, ub2, uW3, ub3, iW1, ib1, iW2, ib2, iW3, ib3):
    bf = jnp.bfloat16
    uw = (uW1.astype(bf), ub1.reshape(1, H1).astype(bf),
          uW2.astype(bf), ub2.reshape(1, H2).astype(bf),
          uW3.astype(bf), ub3.reshape(1, OUT))
    iw = (iW1.astype(bf), ib1.reshape(1, H1).astype(bf),
          iW2.astype(bf), ib2.reshape(1, H2).astype(bf),
          iW3.astype(bf), ib3.reshape(1, OUT))
    u_emb = _sc_gather(user_table, user_idx.astype(jnp.int32), B)
    i_emb = _sc_gather(item_table, item_idx.astype(jnp.int32), B)
    user_out = _tc_tower(u_emb, user_feats.T.astype(bf), uw, B)
    item_out = _tc_tower(i_emb, item_feats.T.astype(bf), iw, B)
    return (user_out, item_out)


# speedup vs baseline: 1.0551x; 1.0551x over previous
import jax
import jax.numpy as jnp
from jax.experimental import pallas as pl
from jax.experimental.pallas import tpu as pltpu
from jax.experimental.pallas import tpu_sc as plsc

B = 16384
E = 128
FEAT = 64
H1, H2 = 512, 256
OUT = 128

GATHER_WINDOW = 256
BB = 4096


def _make_sc_gather(n):
    mesh = plsc.VectorSubcoreMesh(core_axis_name="c", subcore_axis_name="s")
    out_type = jax.ShapeDtypeStruct((n, E), jnp.float32)

    @pl.kernel(out_type=out_type, mesh=mesh)
    def gather_kernel(t_hbm, i_hbm, o_hbm):
        def body(i_vmem, o_vmem):
            pltpu.sync_copy(t_hbm.at[i_vmem.at[0]], o_vmem)

        pltpu.emit_pipeline(
            body,
            grid=(n // GATHER_WINDOW,),
            in_specs=[pl.BlockSpec((1, GATHER_WINDOW), lambda i: (0, i))],
            out_specs=[pl.BlockSpec((GATHER_WINDOW, E), lambda i: (i, 0))],
            core_axis_name=("c", "s"),
            dimension_semantics=(pltpu.PARALLEL,),
        )(i_hbm, o_hbm)

    return gather_kernel


_GATHER = _make_sc_gather(B)


def _sc_gather(table, idx, n):
    return _GATHER(table, idx.reshape(1, n))


def _tower_body(emb, feat_t, W1, b1, W2, b2, W3, b3, out_ref):
    zero = jnp.bfloat16(0.0)
    xe = emb[...].astype(jnp.bfloat16)
    xft = feat_t[...].astype(jnp.bfloat16)
    x = jnp.concatenate([xe, xft.T], axis=-1)
    h = jnp.dot(x, W1[...], preferred_element_type=jnp.float32)
    h = jnp.maximum(h.astype(jnp.bfloat16) + b1[...], zero)
    h = jnp.dot(h, W2[...], preferred_element_type=jnp.float32)
    h = jnp.maximum(h.astype(jnp.bfloat16) + b2[...], zero)
    o = jnp.dot(h, W3[...], preferred_element_type=jnp.float32) + b3[...]
    ss = jnp.sum(o * o, axis=-1, keepdims=True)
    out_ref[...] = o * jax.lax.rsqrt(jnp.maximum(ss, 1e-24))


def _tc_tower(emb, feats_t, weights, n):
    nb = n // BB
    bspec = lambda shape: pl.BlockSpec(shape, lambda i: (i, 0))
    tspec = lambda shape: pl.BlockSpec(shape, lambda i: (0, i))
    wspec = lambda shape: pl.BlockSpec(shape, lambda i: (0, 0))
    in_specs = [
        bspec((BB, E)), tspec((FEAT, BB)),
        wspec((E + FEAT, H1)), wspec((1, H1)),
        wspec((H1, H2)), wspec((1, H2)),
        wspec((H2, OUT)), wspec((1, OUT)),
    ]
    return pl.pallas_call(
        _tower_body,
        grid=(nb,),
        in_specs=in_specs,
        out_specs=bspec((BB, OUT)),
        out_shape=jax.ShapeDtypeStruct((n, OUT), jnp.float32),
    )(emb, feats_t, *weights)


def kernel(user_idx, user_feats, item_idx, item_feats, user_table, item_table,
           uW1, ub1, uW2, ub2, uW3, ub3, iW1, ib1, iW2, ib2, iW3, ib3):
    bf = jnp.bfloat16
    uw = (uW1.astype(bf), ub1.reshape(1, H1).astype(bf),
          uW2.astype(bf), ub2.reshape(1, H2).astype(bf),
          uW3.astype(bf), ub3.reshape(1, OUT))
    iw = (iW1.astype(bf), ib1.reshape(1, H1).astype(bf),
          iW2.astype(bf), ib2.reshape(1, H2).astype(bf),
          iW3.astype(bf), ib3.reshape(1, OUT))
    u_emb = _sc_gather(user_table, user_idx.astype(jnp.int32), B)
    i_emb = _sc_gather(item_table, item_idx.astype(jnp.int32), B)
    user_out = _tc_tower(u_emb, user_feats.T, uw, B)
    item_out = _tc_tower(i_emb, item_feats.T, iw, B)
    return (user_out, item_out)
